# baseline (speedup 1.0000x reference)
import functools

import jax
import jax.numpy as jnp
from jax import lax
from jax.experimental import pallas as pl
from jax.experimental.pallas import tpu as pltpu
from jax.experimental.pallas import tpu_sc as plsc

C = 26
V = 100000
DM = 32
L = 20
NC, NS = 2, 16
NW = NC * NS
LANES = 16

NBC = 16
LW = 2
NWAVE = L // LW
RW = LW * NBC
IC = RW * C
GS = 104
NG = IC // GS


NB = 3


def _body(batch, x_hbm, tables_hbm, out_hbm, xv, idxv, rows, outv, sem0,
          sem1, sem2):
    wid = lax.axis_index("s") * NC + lax.axis_index("c")
    b_per_w = batch // NW
    nchunks = b_per_w // NBC
    sems = (sem0, sem1, sem2)

    def chunk(g, carry):
        b0 = wid * b_per_w + g * NBC
        pltpu.sync_copy(x_hbm.at[:, :, pl.ds(b0, NBC)], xv)

        def mkidx_wave(w):
            def mkidx(t, c2):
                c = t // LW
                dl = t - c * LW
                idxv[w % NB, pl.ds(t * LANES, LANES)] = (
                    xv[c, w * LW + dl, :] + c * V
                )
                return c2
            lax.fori_loop(0, IC // LANES, mkidx, 0)

        def fire(w):
            return [
                pltpu.async_copy(
                    tables_hbm.at[idxv.at[w % NB, pl.ds(j * GS, GS)]],
                    rows.at[w % NB, pl.ds(j * GS, GS)],
                    sems[w % NB],
                )
                for j in range(NG)
            ]

        inflight = []
        for w in range(NB - 1):
            mkidx_wave(w)
            inflight.append(fire(w))
        for w in range(NWAVE):
            if w + NB - 1 < NWAVE:
                mkidx_wave(w + NB - 1)
                inflight.append(fire(w + NB - 1))
            for cp in inflight.pop(0):
                cp.wait()

            def srow(q, c2):
                dl = q // NBC
                db = q - dl * NBC
                a0 = rows[w % NB, q, pl.ds(0, LANES)]
                a1 = rows[w % NB, q, pl.ds(LANES, LANES)]
                for c in range(1, C):
                    a0 = a0 + rows[w % NB, q + RW * c, pl.ds(0, LANES)]
                    a1 = a1 + rows[w % NB, q + RW * c, pl.ds(LANES, LANES)]
                outv[db, w * LW + dl, pl.ds(0, LANES)] = a0
                outv[db, w * LW + dl, pl.ds(LANES, LANES)] = a1
                return c2
            lax.fori_loop(0, RW, srow, 0)

        pltpu.sync_copy(outv, out_hbm.at[pl.ds(b0, NBC)])
        return carry

    lax.fori_loop(0, nchunks, chunk, 0)


@jax.jit
def kernel(x, tables):
    B, sl, c = x.shape
    assert c == C and sl == L and tables.shape == (C, V, DM)
    assert B % (NW * NBC) == 0

    xt = jnp.transpose(x, (2, 1, 0))
    tables_flat = tables.reshape(C * V, DM)

    mesh = plsc.VectorSubcoreMesh(core_axis_name="c", subcore_axis_name="s")
    call = pl.kernel(
        functools.partial(_body, B),
        out_type=jax.ShapeDtypeStruct((B, L, DM), jnp.float32),
        mesh=mesh,
        compiler_params=pltpu.CompilerParams(use_tc_tiling_on_sc=False),
        scratch_types=[
            pltpu.VMEM((C, L, NBC), jnp.int32),
            pltpu.VMEM((NB, IC), jnp.int32),
            pltpu.VMEM((NB, IC, DM), jnp.float32),
            pltpu.VMEM((NBC, L, DM), jnp.float32),
            pltpu.SemaphoreType.DMA,
            pltpu.SemaphoreType.DMA,
            pltpu.SemaphoreType.DMA,
        ],
    )
    return call(xt, tables_flat)
